# Initial kernel scaffold; baseline (speedup 1.0000x reference)
import jax
import jax.numpy as jnp
from jax.experimental import pallas as pl


def kernel(x, ln_gamma, ln_beta, W_enc, b_enc, W_out, b_out, expert_keys):
    raise NotImplementedError("write your pallas kernel here")


# speedup vs baseline: 2.1373x; 2.1373x over previous
import functools

import jax
import jax.numpy as jnp
from jax.experimental import pallas as pl
from jax.experimental.pallas import tpu as pltpu

N_TOK = 8192
D_MODEL = 2048
LATENT = 512
N_EXPERTS = 16
TOP_K = 2
DIV_LAMBDA = 0.1
KL_W = 0.01
ALIGN_W = 0.1
DIV_W = 0.1
LN_EPS = 1e-5

TILE = 256
GRID = N_TOK // TILE


def _dot(a, b, dims):
    return jax.lax.dot_general(a, b, (dims, ((), ())),
                               preferred_element_type=jnp.float32)


def _router_kernel(x_ref, eps_ref, g_ref, bln_ref, we_ref, be_ref, wo_ref,
                   bo_ref, ek_ref,
                   rw_ref, loss_ref, idx_ref, sc_ref, zd_ref,
                   wkt_acc, snk_acc, tr_acc, kl_acc):
    i = pl.program_id(0)

    @pl.when(i == 0)
    def _init():
        wkt_acc[...] = jnp.zeros_like(wkt_acc)
        snk_acc[...] = jnp.zeros_like(snk_acc)
        tr_acc[...] = jnp.zeros_like(tr_acc)
        kl_acc[...] = jnp.zeros_like(kl_acc)

    x = x_ref[...]
    m = jnp.mean(x, axis=-1, keepdims=True)
    xc = x - m
    v = jnp.mean(xc * xc, axis=-1, keepdims=True)
    hn = g_ref[...] * xc / jnp.sqrt(v + LN_EPS) + bln_ref[...]
    h = hn * jax.nn.sigmoid(hn)

    ml = _dot(h, we_ref[...], (((1,), (1,)))) + be_ref[...]
    mu = ml[:, :LATENT]
    lv = ml[:, LATENT:]
    std = jnp.exp(0.5 * lv)
    z = mu + eps_ref[...] * std

    ek = ek_ref[...]
    scores = _dot(mu, ek, (((1,), (1,))))
    sc_ref[...] = scores

    mx = jnp.max(scores, axis=1, keepdims=True)
    e = jnp.exp(scores - mx)
    sm = e / jnp.sum(e, axis=1, keepdims=True)
    wv = _dot(sm, ek, (((1,), (0,))))
    zd_ref[...] = _dot(wv, wo_ref[...], (((1,), (1,)))) + bo_ref[...]

    iota = jax.lax.broadcasted_iota(jnp.int32, scores.shape, 1)
    v1 = mx
    i1 = jnp.min(jnp.where(scores == v1, iota, N_EXPERTS), axis=1,
                 keepdims=True)
    masked = jnp.where(iota == i1, -jnp.inf, scores)
    v2 = jnp.max(masked, axis=1, keepdims=True)
    i2 = jnp.min(jnp.where(masked == v2, iota, N_EXPERTS), axis=1,
                 keepdims=True)
    idx_ref[...] = jnp.concatenate([i1, i2], axis=1)
    b = jnp.exp(v2 - v1)
    rw_ref[...] = jnp.concatenate([1.0 / (1.0 + b), b / (1.0 + b)], axis=1)

    rs = jnp.where(iota == i1, v1, jnp.where(iota == i2, v2, 0.0))
    rmx = jnp.maximum(v1, 0.0)
    re = jnp.exp(rs - rmx)
    p = re / jnp.sum(re, axis=1, keepdims=True)

    wkt_acc[...] += _dot(p, z, (((0,), (0,))))

    nrm = jnp.sqrt(jnp.sum(mu * mu, axis=1, keepdims=True))
    nk = mu / jnp.clip(nrm, 1e-12, None)
    snk_acc[...] += jnp.sum(nk, axis=0, keepdims=True)
    tr_acc[...] += jnp.sum(nk * nk).reshape(1, 1)
    kl_acc[...] += jnp.sum(1.0 + lv - mu * mu - jnp.exp(lv)).reshape(1, 1)

    @pl.when(i == GRID - 1)
    def _finish():
        s = snk_acc[...]
        ssq = jnp.sum(s * s)
        tr = tr_acc[...][0, 0]
        mu_off = (ssq - tr) / (N_TOK * (N_TOK - 1))

        eknrm = jnp.sqrt(jnp.sum(ek * ek, axis=1, keepdims=True))
        nek = ek / jnp.clip(eknrm, 1e-12, None)
        sim = _dot(nek, nek, (((1,), (1,))))
        eye = (jax.lax.broadcasted_iota(jnp.int32, sim.shape, 0)
               == jax.lax.broadcasted_iota(jnp.int32, sim.shape, 1))
        ek_off = (jnp.sum(sim) - jnp.sum(jnp.where(eye, sim, 0.0))) / (
            N_EXPERTS * (N_EXPERTS - 1))
        div_loss = DIV_LAMBDA * (mu_off + ek_off)

        kl = -0.5 * kl_acc[...][0, 0] / N_TOK
        sim_loss = jnp.mean(jnp.abs(ek - wkt_acc[...]))
        loss_ref[...] = (DIV_W * div_loss + KL_W * kl
                         + ALIGN_W * sim_loss).reshape(1, 1)


@jax.jit
def kernel(x, ln_gamma, ln_beta, W_enc, b_enc, W_out, b_out, expert_keys):
    eps = jax.random.normal(jax.random.key(42), (N_TOK, LATENT),
                            dtype=jnp.float32)

    full = lambda *shape: pl.BlockSpec(shape, lambda i: (0,) * len(shape))
    tiled = lambda cols: pl.BlockSpec((TILE, cols), lambda i: (i, 0))

    out_shapes = (
        jax.ShapeDtypeStruct((N_TOK, TOP_K), jnp.float32),
        jax.ShapeDtypeStruct((1, 1), jnp.float32),
        jax.ShapeDtypeStruct((N_TOK, TOP_K), jnp.int32),
        jax.ShapeDtypeStruct((N_TOK, N_EXPERTS), jnp.float32),
        jax.ShapeDtypeStruct((N_TOK, D_MODEL), jnp.float32),
    )
    out_specs = (tiled(TOP_K), full(1, 1), tiled(TOP_K), tiled(N_EXPERTS),
                 tiled(D_MODEL))
    in_specs = (
        tiled(D_MODEL),
        tiled(LATENT),
        full(D_MODEL),
        full(D_MODEL),
        full(2 * LATENT, D_MODEL),
        full(2 * LATENT),
        full(D_MODEL, LATENT),
        full(D_MODEL),
        full(N_EXPERTS, LATENT),
    )
    scratch = [
        pltpu.VMEM((N_EXPERTS, LATENT), jnp.float32),
        pltpu.VMEM((1, LATENT), jnp.float32),
        pltpu.VMEM((1, 1), jnp.float32),
        pltpu.VMEM((1, 1), jnp.float32),
    ]
    rw, loss, idx, scores, zd = pl.pallas_call(
        _router_kernel,
        grid=(GRID,),
        in_specs=in_specs,
        out_specs=out_specs,
        out_shape=out_shapes,
        scratch_shapes=scratch,
        compiler_params=pltpu.CompilerParams(
            dimension_semantics=("arbitrary",)),
    )(x, eps, ln_gamma, ln_beta, W_enc, b_enc, W_out, b_out, expert_keys)
    return (rw, loss.reshape(()), idx, scores, zd)
